# baseline (speedup 1.0000x reference)
import functools
import jax
import jax.numpy as jnp
from jax.experimental import pallas as pl
from jax.experimental.pallas import tpu as pltpu


def _mm_body(x_ref, w_ref, o_ref):
    o_ref[...] = jnp.dot(x_ref[...], w_ref[...],
                         preferred_element_type=jnp.float32)


def _matmul(x, w, block_rows=2000):
    n, d = x.shape
    _, m = w.shape
    grid = (n // block_rows,) if n % block_rows == 0 else None
    if grid is None:
        return pl.pallas_call(
            _mm_body,
            out_shape=jax.ShapeDtypeStruct((n, m), jnp.float32),
        )(x, w)
    return pl.pallas_call(
        _mm_body,
        grid=grid,
        in_specs=[
            pl.BlockSpec((block_rows, d), lambda i: (i, 0)),
            pl.BlockSpec((d, m), lambda i: (0, 0)),
        ],
        out_specs=pl.BlockSpec((block_rows, m), lambda i: (i, 0)),
        out_shape=jax.ShapeDtypeStruct((n, m), jnp.float32),
    )(x, w)


def _gat_conv(x, src, dst, W, att_src, att_dst, bias):
    N = x.shape[0]
    H, C = att_src.shape
    h = _matmul(x, W).reshape(N, H, C)
    a_src = jnp.sum(h * att_src[None, :, :], axis=-1)
    a_dst = jnp.sum(h * att_dst[None, :, :], axis=-1)
    alpha = jax.nn.leaky_relu(a_src[src] + a_dst[dst], negative_slope=0.2)
    amax = jax.ops.segment_max(alpha, dst, num_segments=N)
    amax = jnp.where(jnp.isfinite(amax), amax, 0.0)
    ex = jnp.exp(alpha - amax[dst])
    denom = jax.ops.segment_sum(ex, dst, num_segments=N)
    coef = ex / (denom[dst] + 1e-16)
    out = jax.ops.segment_sum(h[src] * coef[:, :, None], dst, num_segments=N)
    return out.mean(axis=1) + bias


def _bn_relu(x, gamma, beta, eps=1e-5):
    mu = x.mean(axis=0)
    var = x.var(axis=0)
    return jax.nn.relu((x - mu) / jnp.sqrt(var + eps) * gamma + beta)


def kernel(x, edge_index, doc_data, W1, att_src1, att_dst1, b1, g1, beta1,
           W2, att_src2, att_dst2, b2, g2, beta2,
           W3, att_src3, att_dst3, b3, Wd, bd):
    N = x.shape[0]
    loop = jnp.arange(N, dtype=edge_index.dtype)
    src = jnp.concatenate([edge_index[0], loop])
    dst = jnp.concatenate([edge_index[1], loop])
    h = _gat_conv(x, src, dst, W1, att_src1, att_dst1, b1)
    h = _bn_relu(h, g1, beta1)
    h = _gat_conv(h, src, dst, W2, att_src2, att_dst2, b2)
    h = _bn_relu(h, g2, beta2)
    h = _gat_conv(h, src, dst, W3, att_src3, att_dst3, b3)
    doc = _matmul(doc_data, Wd.T) + bd
    return (h, doc)
